# baseline (speedup 1.0000x reference)
import functools

import jax
import jax.numpy as jnp
from jax import lax
from jax.experimental import pallas as pl
from jax.experimental.pallas import tpu as pltpu
from jax.experimental.pallas import tpu_sc as plsc

_TB = 512


def _vq_body(x_ref, e_ref, ind_ref, diff_ref, e2_ref):
    xb = x_ref[...]
    e = e_ref[...]
    C = e.shape[1]

    @pl.when(pl.program_id(0) == 0)
    def _compute_e2():
        e2_ref[...] = jnp.sum(e * e, axis=0, keepdims=True)

    mm2 = jax.lax.dot_general(
        xb * (-2.0), e, (((1,), (0,)), ((), ())),
        preferred_element_type=jnp.float32)
    x2 = jnp.sum(xb * xb, axis=1, keepdims=True)
    score = mm2 + e2_ref[...]
    minv = jnp.min(score, axis=1, keepdims=True)
    iota = jax.lax.broadcasted_iota(jnp.int32, score.shape, 1)
    idx = jnp.min(jnp.where(score == minv, iota, C), axis=1)
    ind_ref[0, 0, :] = idx
    part = (jnp.sum(x2) + jnp.sum(minv)).reshape(1, 1)

    @pl.when(pl.program_id(0) == 0)
    def _init():
        diff_ref[...] = part

    @pl.when(pl.program_id(0) != 0)
    def _acc():
        diff_ref[...] += part


def _tc_stage(x, embed):
    N, D = x.shape
    C = embed.shape[1]
    nb = N // _TB
    ind, diff = pl.pallas_call(
        _vq_body,
        grid=(nb,),
        in_specs=[
            pl.BlockSpec((_TB, D), lambda i: (i, 0)),
            pl.BlockSpec((D, C), lambda i: (0, 0)),
        ],
        out_specs=[
            pl.BlockSpec((1, 1, _TB), lambda i: (i, 0, 0)),
            pl.BlockSpec((1, 1), lambda i: (0, 0)),
        ],
        out_shape=[
            jax.ShapeDtypeStruct((nb, 1, _TB), jnp.int32),
            jax.ShapeDtypeStruct((1, 1), jnp.float32),
        ],
        scratch_shapes=[pltpu.VMEM((1, C), jnp.float32)],
    )(x, embed)
    return ind.reshape(N), diff


def _make_sc_gather(V, D, B):
    info = plsc.get_sparse_core_info()
    nw = info.num_cores * info.num_subcores
    b_per_w = B // nw
    mesh = plsc.VectorSubcoreMesh(core_axis_name="c", subcore_axis_name="s")

    @functools.partial(
        pl.kernel, mesh=mesh,
        out_type=jax.ShapeDtypeStruct((B, D), jnp.float32),
        scratch_types=[
            pltpu.VMEM((b_per_w,), jnp.int32),
            pltpu.VMEM((b_per_w, D), jnp.float32),
            pltpu.SemaphoreType.DMA,
        ],
    )
    def gather(table_hbm, idx_hbm, out_hbm, idx_v, rows_v, sem):
        wid = lax.axis_index("s") * info.num_cores + lax.axis_index("c")
        base = wid * b_per_w
        pltpu.sync_copy(idx_hbm.at[pl.ds(base, b_per_w)], idx_v)
        pltpu.async_copy(table_hbm.at[idx_v], rows_v, sem).wait()
        pltpu.sync_copy(rows_v, out_hbm.at[pl.ds(base, b_per_w)])

    return gather


def kernel(input, embed):
    x = input.reshape(-1, embed.shape[0])
    N, D = x.shape
    C = embed.shape[1]
    ind, diff = _tc_stage(x, embed)
    table = jnp.pad(embed.T, ((0, 0), (0, 128 - D)))
    q = _make_sc_gather(C, 128, N)(table, ind)[:, :D]
    scale = 1.0 / (N * D)
    return (q.reshape(input.shape),
            (diff[0, 0] * scale).astype(jnp.float32).reshape(()),
            ind.reshape(input.shape[:-1]))


# speedup vs baseline: 1.0720x; 1.0720x over previous
import functools

import jax
import jax.numpy as jnp
from jax import lax
from jax.experimental import pallas as pl
from jax.experimental.pallas import tpu as pltpu
from jax.experimental.pallas import tpu_sc as plsc

_TB = 512


def _vq_body(x_ref, e_ref, ind_ref, diff_ref, e2_ref):
    xb = x_ref[...]
    e = e_ref[...]
    C = e.shape[1]

    @pl.when(pl.program_id(0) == 0)
    def _compute_e2():
        e2_ref[...] = jnp.sum(e * e, axis=0, keepdims=True)

    mm = jax.lax.dot_general(
        xb, e, (((1,), (0,)), ((), ())),
        preferred_element_type=jnp.float32)
    x2 = jnp.sum(xb * xb, axis=1, keepdims=True)
    dist2 = x2 - 2.0 * mm + e2_ref[...]
    minv = jnp.min(dist2, axis=1, keepdims=True)
    iota = jax.lax.broadcasted_iota(jnp.int32, dist2.shape, 1)
    idx = jnp.min(jnp.where(dist2 == minv, iota, C), axis=1)
    ind_ref[0, 0, :] = idx
    part = jnp.sum(minv).reshape(1, 1)

    @pl.when(pl.program_id(0) == 0)
    def _init():
        diff_ref[...] = part

    @pl.when(pl.program_id(0) != 0)
    def _acc():
        diff_ref[...] += part


def _tc_stage(x, embed):
    N, D = x.shape
    C = embed.shape[1]
    nb = N // _TB
    ind, diff = pl.pallas_call(
        _vq_body,
        grid=(nb,),
        in_specs=[
            pl.BlockSpec((_TB, D), lambda i: (i, 0)),
            pl.BlockSpec((D, C), lambda i: (0, 0)),
        ],
        out_specs=[
            pl.BlockSpec((1, 1, _TB), lambda i: (i, 0, 0)),
            pl.BlockSpec((1, 1), lambda i: (0, 0)),
        ],
        out_shape=[
            jax.ShapeDtypeStruct((nb, 1, _TB), jnp.int32),
            jax.ShapeDtypeStruct((1, 1), jnp.float32),
        ],
        scratch_shapes=[pltpu.VMEM((1, C), jnp.float32)],
    )(x, embed)
    return ind.reshape(N), diff


def _make_sc_gather(V, D, B):
    info = plsc.get_sparse_core_info()
    nw = info.num_cores * info.num_subcores
    b_per_w = B // nw
    mesh = plsc.VectorSubcoreMesh(core_axis_name="c", subcore_axis_name="s")

    @functools.partial(
        pl.kernel, mesh=mesh,
        out_type=jax.ShapeDtypeStruct((B, D), jnp.float32),
        scratch_types=[
            pltpu.VMEM((b_per_w,), jnp.int32),
            pltpu.VMEM((b_per_w, D), jnp.float32),
            pltpu.SemaphoreType.DMA,
        ],
    )
    def gather(table_hbm, idx_hbm, out_hbm, idx_v, rows_v, sem):
        wid = lax.axis_index("s") * info.num_cores + lax.axis_index("c")
        base = wid * b_per_w
        pltpu.sync_copy(idx_hbm.at[pl.ds(base, b_per_w)], idx_v)
        pltpu.async_copy(table_hbm.at[idx_v], rows_v, sem).wait()
        pltpu.sync_copy(rows_v, out_hbm.at[pl.ds(base, b_per_w)])

    return gather


def kernel(input, embed):
    x = input.reshape(-1, embed.shape[0])
    N, D = x.shape
    C = embed.shape[1]
    ind, diff = _tc_stage(x, embed)
    table = jnp.pad(embed.T, ((0, 0), (0, 128 - D)))
    q = _make_sc_gather(C, 128, N)(table, ind)[:, :D]
    scale = 1.0 / (N * D)
    return (q.reshape(input.shape),
            (diff[0, 0] * scale).astype(jnp.float32).reshape(()),
            ind.reshape(input.shape[:-1]))
